# baseline (speedup 1.0000x reference)
import functools

import jax
import jax.numpy as jnp
import numpy as np
from jax import lax
from jax.experimental import pallas as pl
from jax.experimental.pallas import tpu as pltpu
from jax.experimental.pallas import tpu_sc as plsc

NC = 2
NS = 16
L = 16
NW = NC * NS

IDX_W = 128
SUB = 13
CHUNK = SUB * IDX_W
GROUP = 4
GSH = 2


@functools.lru_cache(maxsize=None)
def _build(B, F, V, D):
    TOT = B * F
    assert TOT % (NW * CHUNK) == 0
    W = GROUP * D
    DSH = D.bit_length() - 1
    per_w = TOT // NW
    n_chunks = per_w // CHUNK
    out_rows = CHUNK // GROUP

    mesh = plsc.VectorSubcoreMesh(core_axis_name="c", subcore_axis_name="s")

    @functools.partial(
        pl.kernel,
        mesh=mesh,
        out_type=jax.ShapeDtypeStruct((TOT // GROUP, W), jnp.float32),
        scratch_types=[
            pltpu.VMEM((CHUNK,), jnp.int32),
            pltpu.VMEM((CHUNK,), jnp.int32),
            pltpu.VMEM((CHUNK,), jnp.int32),
            pltpu.VMEM((2, IDX_W, W), jnp.float32),
            pltpu.VMEM((out_rows, W), jnp.float32),
            pltpu.SemaphoreType.DMA,
        ],
        compiler_params=pltpu.CompilerParams(needs_layout_passes=False),
    )
    def gather_kernel(x_hbm, tab_hbm, offs_hbm, out_hbm, idx_v, sub_v,
                      offs_v, gath_v, out_v, sem):
        wid = lax.axis_index("s") * NC + lax.axis_index("c")
        pltpu.sync_copy(offs_hbm, offs_v)
        lane = lax.iota(jnp.int32, L)
        lane_div = lax.shift_right_logical(lane, GSH)
        lane_col = lax.shift_left(lane & (GROUP - 1), DSH)

        def chunk_body(c, carry):
            base = pl.multiple_of((wid * n_chunks + c) * CHUNK, 8)
            pltpu.sync_copy(x_hbm.at[pl.ds(base, CHUNK)], idx_v)
            for t in range(CHUNK // L):
                sl = pl.ds(t * L, L)
                flat = idx_v[sl] + offs_v[sl]
                idx_v[sl] = lax.shift_right_logical(flat, GSH)
                sub_v[sl] = lax.shift_left(flat & (GROUP - 1), DSH)

            def fire(k, buf):
                return pltpu.async_copy(
                    tab_hbm.at[idx_v.at[pl.ds(k * IDX_W, IDX_W)]],
                    gath_v.at[buf],
                    sem,
                )

            cp = fire(0, 0)
            for k in range(SUB):
                cp.wait()
                if k + 1 < SUB:
                    cp = fire(k + 1, (k + 1) % 2)
                gbuf = gath_v.at[k % 2]

                def extract(t, carry):
                    row_in = t * L + lane
                    col_base = sub_v[pl.ds(k * IDX_W + t * L, L)]
                    orow = k * (IDX_W // GROUP) + t * (L // GROUP) + lane_div
                    for d in range(D):
                        vals = plsc.load_gather(gbuf, [row_in, col_base + d])
                        plsc.store_scatter(out_v, [orow, lane_col + d], vals)
                    return carry

                lax.fori_loop(0, IDX_W // L, extract, None)
            obase = pl.multiple_of(base // GROUP, 8)
            pltpu.sync_copy(out_v, out_hbm.at[pl.ds(obase, out_rows)])
            return carry

        lax.fori_loop(0, n_chunks, chunk_body, None)

    return gather_kernel


@functools.lru_cache(maxsize=None)
def _field_offsets(F, V):
    offs = (np.arange(CHUNK, dtype=np.int64) % F) * V
    return jnp.asarray(offs.astype(np.int32))


def kernel(x_cat, tables):
    B, F = x_cat.shape
    _, V, D = tables.shape
    x_flat = x_cat.reshape(-1)
    tab = tables.reshape(F * V // GROUP, GROUP * D)
    out = _build(B, F, V, D)(x_flat, tab, _field_offsets(F, V))
    return out.reshape(B, F * D)


# speedup vs baseline: 2.1404x; 2.1404x over previous
import functools

import jax
import jax.numpy as jnp
from jax import lax
from jax.experimental import pallas as pl
from jax.experimental.pallas import tpu as pltpu
from jax.experimental.pallas import tpu_sc as plsc

NC = 2
NS = 16
L = 16
NW = NC * NS
HALF = 2


@functools.lru_cache(maxsize=None)
def _build(B, F, V, D):
    n_tasks = F * D
    DSH = D.bit_length() - 1
    assert n_tasks % NW == 0 and (1 << DSH) == D
    per_w = n_tasks // NW
    BH = B // HALF

    mesh = plsc.VectorSubcoreMesh(core_axis_name="c", subcore_axis_name="s")

    @functools.partial(
        pl.kernel,
        mesh=mesh,
        out_type=jax.ShapeDtypeStruct((n_tasks * B,), jnp.float32),
        scratch_types=[
            pltpu.VMEM((V,), jnp.float32),
            pltpu.VMEM((B,), jnp.int32),
            pltpu.VMEM((BH,), jnp.float32),
        ],
        compiler_params=pltpu.CompilerParams(use_tc_tiling_on_sc=False, needs_layout_passes=False),
    )
    def gather_kernel(xt_hbm, tab_hbm, out_hbm, src_v, x_v, out_v):
        wid = lax.axis_index("s") * NC + lax.axis_index("c")
        t0 = wid * per_w
        j_first = lax.shift_right_logical(t0, DSH)
        pltpu.sync_copy(xt_hbm.at[pl.ds(pl.multiple_of(j_first * B, 8), B)],
                        x_v)

        def task_body(tau, j_prev):
            tid = t0 + tau
            j = lax.shift_right_logical(tid, DSH)
            d = tid - lax.shift_left(j, DSH)

            @pl.when(j != j_prev)
            def _():
                pltpu.sync_copy(
                    xt_hbm.at[pl.ds(pl.multiple_of(j * B, 8), B)], x_v)

            tbase = pl.multiple_of(tid * V, 8)
            pltpu.sync_copy(tab_hbm.at[pl.ds(tbase, V)], src_v)
            for h in range(HALF):

                def gather16(i, carry):
                    sl = pl.ds(h * BH + i * L, L)
                    out_v[pl.ds(i * L, L)] = plsc.load_gather(
                        src_v, [x_v[sl]])
                    return carry

                lax.fori_loop(0, BH // L, gather16, None)
                obase = pl.multiple_of(tid * B + h * BH, 8)
                pltpu.sync_copy(out_v, out_hbm.at[pl.ds(obase, BH)])
            return j

        lax.fori_loop(0, per_w, task_body, j_first)

    return gather_kernel


def kernel(x_cat, tables):
    B, F = x_cat.shape
    _, V, D = tables.shape
    xt = x_cat.T.reshape(-1)
    tabt = tables.transpose(0, 2, 1).reshape(-1)
    out = _build(B, F, V, D)(xt, tabt)
    return out.reshape(F * D, B).T.reshape(B, F * D)


# speedup vs baseline: 2.2654x; 1.0584x over previous
import functools

import jax
import jax.numpy as jnp
from jax import lax
from jax.experimental import pallas as pl
from jax.experimental.pallas import tpu as pltpu
from jax.experimental.pallas import tpu_sc as plsc

NC = 2
NS = 16
L = 16
NW = NC * NS
HALF = 2


@functools.lru_cache(maxsize=None)
def _build(B, F, V, D):
    n_tasks = F * D
    DSH = D.bit_length() - 1
    assert n_tasks % NW == 0 and (1 << DSH) == D
    per_w = n_tasks // NW
    BH = B // HALF

    mesh = plsc.VectorSubcoreMesh(core_axis_name="c", subcore_axis_name="s")

    @functools.partial(
        pl.kernel,
        mesh=mesh,
        out_type=jax.ShapeDtypeStruct((n_tasks * B,), jnp.float32),
        scratch_types=[
            pltpu.VMEM((V,), jnp.float32),
            pltpu.VMEM((B,), jnp.int32),
            pltpu.VMEM((BH,), jnp.float32),
        ],
        compiler_params=pltpu.CompilerParams(use_tc_tiling_on_sc=False, needs_layout_passes=False),
    )
    def gather_kernel(xt_hbm, tab_hbm, out_hbm, src_v, x_v, out_v):
        wid = lax.axis_index("s") * NC + lax.axis_index("c")
        t0 = wid * per_w
        j_first = lax.shift_right_logical(t0, DSH)
        pltpu.sync_copy(xt_hbm.at[pl.ds(pl.multiple_of(j_first * B, 8), B)],
                        x_v)

        def task_body(tau, j_prev):
            tid = t0 + tau
            j = lax.shift_right_logical(tid, DSH)
            d = tid - lax.shift_left(j, DSH)

            @pl.when(j != j_prev)
            def _():
                pltpu.sync_copy(
                    xt_hbm.at[pl.ds(pl.multiple_of(j * B, 8), B)], x_v)

            tbase = pl.multiple_of(tid * V, 8)
            pltpu.sync_copy(tab_hbm.at[pl.ds(tbase, V)], src_v)
            for h in range(HALF):

                def gather64(i, carry):
                    for u in range(4):
                        sl = pl.ds(h * BH + i * 4 * L + u * L, L)
                        out_v[pl.ds(i * 4 * L + u * L, L)] = (
                            plsc.load_gather(src_v, [x_v[sl]]))
                    return carry

                lax.fori_loop(0, BH // (4 * L), gather64, None)
                obase = pl.multiple_of(tid * B + h * BH, 8)
                pltpu.sync_copy(out_v, out_hbm.at[pl.ds(obase, BH)])
            return j

        lax.fori_loop(0, per_w, task_body, j_first)

    return gather_kernel


def kernel(x_cat, tables):
    B, F = x_cat.shape
    _, V, D = tables.shape
    xt = x_cat.T.reshape(-1)
    tabt = tables.transpose(0, 2, 1).reshape(-1)
    out = _build(B, F, V, D)(xt, tabt)
    return out.reshape(F * D, B).T.reshape(B, F * D)


# speedup vs baseline: 2.3040x; 1.0170x over previous
import functools

import jax
import jax.numpy as jnp
from jax import lax
from jax.experimental import pallas as pl
from jax.experimental.pallas import tpu as pltpu
from jax.experimental.pallas import tpu_sc as plsc

NC = 2
NS = 16
L = 16
NW = NC * NS
HALF = 4


@functools.lru_cache(maxsize=None)
def _build(B, F, V, D):
    n_tasks = F * D
    DSH = D.bit_length() - 1
    assert n_tasks % NW == 0 and (1 << DSH) == D
    per_w = n_tasks // NW
    BH = B // HALF

    mesh = plsc.VectorSubcoreMesh(core_axis_name="c", subcore_axis_name="s")

    @functools.partial(
        pl.kernel,
        mesh=mesh,
        out_type=jax.ShapeDtypeStruct((n_tasks * B,), jnp.float32),
        scratch_types=[
            pltpu.VMEM((V,), jnp.float32),
            pltpu.VMEM((B,), jnp.int32),
            pltpu.VMEM((2, BH), jnp.float32),
            pltpu.SemaphoreType.DMA,
        ],
        compiler_params=pltpu.CompilerParams(use_tc_tiling_on_sc=False, needs_layout_passes=False),
    )
    def gather_kernel(xt_hbm, tab_hbm, out_hbm, src_v, x_v, out_v, osem):
        wid = lax.axis_index("s") * NC + lax.axis_index("c")
        t0 = wid * per_w
        j_first = lax.shift_right_logical(t0, DSH)
        pltpu.sync_copy(xt_hbm.at[pl.ds(pl.multiple_of(j_first * B, 8), B)],
                        x_v)

        def task_body(tau, j_prev):
            tid = t0 + tau
            j = lax.shift_right_logical(tid, DSH)
            d = tid - lax.shift_left(j, DSH)

            @pl.when(j != j_prev)
            def _():
                pltpu.sync_copy(
                    xt_hbm.at[pl.ds(pl.multiple_of(j * B, 8), B)], x_v)

            tbase = pl.multiple_of(tid * V, 8)
            pltpu.sync_copy(tab_hbm.at[pl.ds(tbase, V)], src_v)
            cps = [None, None]
            for h in range(HALF):
                buf = h % 2
                if cps[buf] is not None:
                    cps[buf].wait()
                obuf = out_v.at[buf]

                def gather64(i, carry):
                    for u in range(4):
                        sl = pl.ds(h * BH + i * 4 * L + u * L, L)
                        obuf[pl.ds(i * 4 * L + u * L, L)] = (
                            plsc.load_gather(src_v, [x_v[sl]]))
                    return carry

                lax.fori_loop(0, BH // (4 * L), gather64, None)
                obase = pl.multiple_of(tid * B + h * BH, 8)
                cps[buf] = pltpu.async_copy(
                    obuf, out_hbm.at[pl.ds(obase, BH)], osem)
            for cp in cps:
                cp.wait()
            return j

        lax.fori_loop(0, per_w, task_body, j_first)

    return gather_kernel


def kernel(x_cat, tables):
    B, F = x_cat.shape
    _, V, D = tables.shape
    xt = x_cat.T.reshape(-1)
    tabt = tables.transpose(0, 2, 1).reshape(-1)
    out = _build(B, F, V, D)(xt, tabt)
    return out.reshape(F * D, B).T.reshape(B, F * D)


# speedup vs baseline: 2.3106x; 1.0029x over previous
import functools

import jax
import jax.numpy as jnp
from jax import lax
from jax.experimental import pallas as pl
from jax.experimental.pallas import tpu as pltpu
from jax.experimental.pallas import tpu_sc as plsc

NC = 2
NS = 16
L = 16
NW = NC * NS
HALF = 4


@functools.lru_cache(maxsize=None)
def _build(B, F, V, D):
    n_tasks = F * D
    DSH = D.bit_length() - 1
    assert n_tasks % NW == 0 and (1 << DSH) == D
    per_w = n_tasks // NW
    BH = B // HALF

    mesh = plsc.VectorSubcoreMesh(core_axis_name="c", subcore_axis_name="s")

    @functools.partial(
        pl.kernel,
        mesh=mesh,
        out_type=jax.ShapeDtypeStruct((n_tasks * B,), jnp.float32),
        scratch_types=[
            pltpu.VMEM((V,), jnp.float32),
            pltpu.VMEM((B,), jnp.int32),
            pltpu.VMEM((2, BH), jnp.float32),
            pltpu.SemaphoreType.DMA,
        ],
        compiler_params=pltpu.CompilerParams(use_tc_tiling_on_sc=False, needs_layout_passes=False),
    )
    def gather_kernel(xt_hbm, tab_hbm, out_hbm, src_v, x_v, out_v, osem):
        wid = lax.axis_index("s") * NC + lax.axis_index("c")
        t0 = wid * per_w
        j_first = lax.shift_right_logical(t0, DSH)
        pltpu.sync_copy(xt_hbm.at[pl.ds(pl.multiple_of(j_first * B, 8), B)],
                        x_v)

        def task_body(tau, j_prev):
            tid = t0 + tau
            j = lax.shift_right_logical(tid, DSH)
            d = tid - lax.shift_left(j, DSH)

            @pl.when(j != j_prev)
            def _():
                pltpu.sync_copy(
                    xt_hbm.at[pl.ds(pl.multiple_of(j * B, 8), B)], x_v)

            tbase = pl.multiple_of(tid * V, 8)
            pltpu.sync_copy(tab_hbm.at[pl.ds(tbase, V)], src_v)
            cps = [None, None]
            for h in range(HALF):
                buf = h % 2
                if cps[buf] is not None:
                    cps[buf].wait()
                obuf = out_v.at[buf]

                def gather128(i, carry):
                    for u in range(8):
                        sl = pl.ds(h * BH + i * 8 * L + u * L, L)
                        obuf[pl.ds(i * 8 * L + u * L, L)] = (
                            plsc.load_gather(src_v, [x_v[sl]]))
                    return carry

                lax.fori_loop(0, BH // (8 * L), gather128, None)
                obase = pl.multiple_of(tid * B + h * BH, 8)
                cps[buf] = pltpu.async_copy(
                    obuf, out_hbm.at[pl.ds(obase, BH)], osem)
            for cp in cps:
                cp.wait()
            return j

        lax.fori_loop(0, per_w, task_body, j_first)

    return gather_kernel


def kernel(x_cat, tables):
    B, F = x_cat.shape
    _, V, D = tables.shape
    xt = x_cat.T.reshape(-1)
    tabt = tables.transpose(0, 2, 1).reshape(-1)
    out = _build(B, F, V, D)(xt, tabt)
    return out.reshape(F * D, B).T.reshape(B, F * D)
